# baseline (speedup 1.0000x reference)
import jax
import jax.numpy as jnp
from jax.experimental import pallas as pl
from jax.experimental.pallas import tpu as pltpu


def _gcn_kernel(adj_ref, x_ref, w_ref, b_ref, out_ref, s_ref):
    i = pl.program_id(0)

    @pl.when(i == 0)
    def _():
        s_ref[...] = jnp.dot(x_ref[...], w_ref[...],
                             preferred_element_type=jnp.float32
                             ).astype(jnp.bfloat16)

    out_ref[...] = jnp.dot(adj_ref[...].astype(jnp.bfloat16), s_ref[...],
                           preferred_element_type=jnp.float32) + b_ref[...]


def kernel(x, adj, W, b):
    n, d_in = x.shape
    d_out = W.shape[1]

    br = 80
    out = pl.pallas_call(
        _gcn_kernel,
        grid=(n // br,),
        in_specs=[
            pl.BlockSpec((br, n), lambda i: (i, 0)),
            pl.BlockSpec((n, d_in), lambda i: (0, 0)),
            pl.BlockSpec((d_in, d_out), lambda i: (0, 0)),
            pl.BlockSpec((1, d_out), lambda i: (0, 0)),
        ],
        out_specs=pl.BlockSpec((br, d_out), lambda i: (i, 0)),
        out_shape=jax.ShapeDtypeStruct((n, d_out), jnp.float32),
        scratch_shapes=[pltpu.VMEM((n, d_out), jnp.bfloat16)],
    )(adj, x, W, b.reshape(1, d_out))
    return out


# speedup vs baseline: 1.3814x; 1.3814x over previous
import jax
import jax.numpy as jnp
from jax.experimental import pallas as pl
from jax.experimental.pallas import tpu as pltpu


def _gcn_kernel(adj_ref, x_ref, w_ref, b_ref, out_ref, s_ref):
    i = pl.program_id(0)

    @pl.when(i == 0)
    def _():
        s_ref[...] = jnp.dot(x_ref[...], w_ref[...],
                             preferred_element_type=jnp.float32)

    out_ref[...] = jnp.dot(adj_ref[...], s_ref[...],
                           preferred_element_type=jnp.float32) + b_ref[...]


def kernel(x, adj, W, b):
    n, d_in = x.shape
    d_out = W.shape[1]

    br = 200
    out = pl.pallas_call(
        _gcn_kernel,
        grid=(n // br,),
        in_specs=[
            pl.BlockSpec((br, n), lambda i: (i, 0)),
            pl.BlockSpec((n, d_in), lambda i: (0, 0)),
            pl.BlockSpec((d_in, d_out), lambda i: (0, 0)),
            pl.BlockSpec((1, d_out), lambda i: (0, 0)),
        ],
        out_specs=pl.BlockSpec((br, d_out), lambda i: (i, 0)),
        out_shape=jax.ShapeDtypeStruct((n, d_out), jnp.float32),
        scratch_shapes=[pltpu.VMEM((n, d_out), jnp.float32)],
    )(adj, x, W, b.reshape(1, d_out))
    return out


# speedup vs baseline: 1.3818x; 1.0003x over previous
import jax
import jax.numpy as jnp
from jax.experimental import pallas as pl
from jax.experimental.pallas import tpu as pltpu


def _gcn_kernel(adj_ref, x_ref, w_ref, b_ref, out_ref, s_ref):
    i = pl.program_id(0)

    @pl.when(i == 0)
    def _():
        s_ref[...] = jnp.dot(x_ref[...], w_ref[...],
                             preferred_element_type=jnp.float32)

    out_ref[...] = jnp.dot(adj_ref[...], s_ref[...],
                           preferred_element_type=jnp.float32) + b_ref[...]


def kernel(x, adj, W, b):
    n, d_in = x.shape
    d_out = W.shape[1]

    br = 400
    out = pl.pallas_call(
        _gcn_kernel,
        grid=(n // br,),
        in_specs=[
            pl.BlockSpec((br, n), lambda i: (i, 0)),
            pl.BlockSpec((n, d_in), lambda i: (0, 0)),
            pl.BlockSpec((d_in, d_out), lambda i: (0, 0)),
            pl.BlockSpec((1, d_out), lambda i: (0, 0)),
        ],
        out_specs=pl.BlockSpec((br, d_out), lambda i: (i, 0)),
        out_shape=jax.ShapeDtypeStruct((n, d_out), jnp.float32),
        scratch_shapes=[pltpu.VMEM((n, d_out), jnp.float32)],
    )(adj, x, W, b.reshape(1, d_out))
    return out


# speedup vs baseline: 1.3825x; 1.0005x over previous
import jax
import jax.numpy as jnp
from jax.experimental import pallas as pl
from jax.experimental.pallas import tpu as pltpu


def _gcn_kernel(adj_ref, x_ref, w_ref, b_ref, out_ref):
    tmp = jnp.dot(adj_ref[...], x_ref[...],
                  preferred_element_type=jnp.float32)
    out_ref[...] = jnp.dot(tmp, w_ref[...],
                           preferred_element_type=jnp.float32) + b_ref[...]


def kernel(x, adj, W, b):
    n, d_in = x.shape
    d_out = W.shape[1]

    br = 400
    out = pl.pallas_call(
        _gcn_kernel,
        grid=(n // br,),
        in_specs=[
            pl.BlockSpec((br, n), lambda i: (i, 0)),
            pl.BlockSpec((n, d_in), lambda i: (0, 0)),
            pl.BlockSpec((d_in, d_out), lambda i: (0, 0)),
            pl.BlockSpec((1, d_out), lambda i: (0, 0)),
        ],
        out_specs=pl.BlockSpec((br, d_out), lambda i: (i, 0)),
        out_shape=jax.ShapeDtypeStruct((n, d_out), jnp.float32),
    )(adj, x, W, b.reshape(1, d_out))
    return out
